# baseline (speedup 1.0000x reference)
import functools

import jax
import jax.numpy as jnp
from jax import lax
from jax.experimental import pallas as pl
from jax.experimental.pallas import tpu as pltpu
from jax.experimental.pallas import tpu_sc as plsc

_BATCH = 16384
_DIM = 64
_LANES = 16
_NUM_WORKERS = 32
_BPW = _BATCH // _NUM_WORKERS
_CHUNK = 128
_NCHUNK = _BPW // _CHUNK


def _vec_sqrt(x):
    i = lax.bitcast_convert_type(x, jnp.int32)
    i = jnp.int32(0x1FBD1DF5) + lax.shift_right_logical(i, 1)
    y = lax.bitcast_convert_type(i, jnp.float32)
    for _ in range(3):
        y = 0.5 * (y + x / y)
    return y


def _tec_body(head, relation, tail, entity, rel_table, out,
              hidx, ridx, tidx, hrows, rrows, trows, outv, sem):
    wid = lax.axis_index("s") * 2 + lax.axis_index("c")
    base = wid * _BPW

    for j in range(_NCHUNK):
        src = pl.ds(base + j * _CHUNK, _CHUNK)
        pltpu.sync_copy(head.at[src], hidx.at[j])
        pltpu.sync_copy(relation.at[src], ridx.at[j])
        pltpu.sync_copy(tail.at[src], tidx.at[j])

    copies = []
    for j in range(_NCHUNK):
        dst = pl.ds(j * _CHUNK, _CHUNK)
        copies.append(pltpu.async_copy(entity.at[hidx.at[j]], hrows.at[dst], sem))
        copies.append(pltpu.async_copy(rel_table.at[ridx.at[j]], rrows.at[dst], sem))
        copies.append(pltpu.async_copy(entity.at[tidx.at[j]], trows.at[dst], sem))
    for cp in copies:
        cp.wait()

    row_iota = lax.iota(jnp.int32, _LANES)

    def body(g, carry):
        res = jnp.zeros((_LANES,), jnp.float32)
        for e in range(_LANES):
            i = g * _LANES + e
            acc = jnp.zeros((_LANES,), jnp.float32)
            for c in range(_DIM // _LANES):
                sl = pl.ds(c * _LANES, _LANES)
                s = hrows[i, sl] + rrows[i, sl] - trows[i, sl]
                acc = acc + s * s
            res = jnp.where(row_iota == e, jnp.sum(acc), res)
        outv[pl.ds(g * _LANES, _LANES)] = _vec_sqrt(res)
        return carry

    lax.fori_loop(0, _BPW // _LANES, body, 0)

    pltpu.sync_copy(outv, out.at[pl.ds(base, _BPW)])


@functools.partial(
    pl.kernel,
    out_type=jax.ShapeDtypeStruct((_BATCH,), jnp.float32),
    mesh=plsc.VectorSubcoreMesh(core_axis_name="c", subcore_axis_name="s"),
    compiler_params=pltpu.CompilerParams(
        needs_layout_passes=False, use_tc_tiling_on_sc=False
    ),
    scratch_types=[
        pltpu.VMEM((_NCHUNK, _CHUNK), jnp.int32),
        pltpu.VMEM((_NCHUNK, _CHUNK), jnp.int32),
        pltpu.VMEM((_NCHUNK, _CHUNK), jnp.int32),
        pltpu.VMEM((_BPW, _DIM), jnp.float32),
        pltpu.VMEM((_BPW, _DIM), jnp.float32),
        pltpu.VMEM((_BPW, _DIM), jnp.float32),
        pltpu.VMEM((_BPW,), jnp.float32),
        pltpu.SemaphoreType.DMA,
    ],
)
def _transe_sc(*args):
    _tec_body(*args)


def kernel(head, relation, tail, entity_table, relation_table):
    return _transe_sc(head, relation, tail, entity_table, relation_table)
